# Initial kernel scaffold; baseline (speedup 1.0000x reference)
import jax
import jax.numpy as jnp
from jax.experimental import pallas as pl


def kernel(x_user, x_movie, edge_index_rates, edge_index_rated_by, lin_user_W, lin_user_b, lin_movie_W, lin_movie_b, bn_user_g, bn_user_beta, bn_user_m, bn_user_v, bn_movie_g, bn_movie_beta, bn_movie_m, bn_movie_v, c1_rates_Wl, c1_rates_bl, c1_rates_Wr, c1_rb_Wl, c1_rb_bl, c1_rb_Wr, c2_rates_Wl, c2_rates_bl, c2_rates_Wr, c2_rb_Wl, c2_rb_bl, c2_rb_Wr):
    raise NotImplementedError("write your pallas kernel here")


# speedup vs baseline: 3.1246x; 3.1246x over previous
import functools

import jax
import jax.numpy as jnp
from jax import lax
from jax.experimental import pallas as pl
from jax.experimental.pallas import tpu as pltpu
from jax.experimental.pallas import tpu_sc as plsc

N = 10000
E = 320000
F = 128
L = 16
NS = 16
EPT = E // NS
CH = 80
NCHUNK = EPT // CH


def _make_agg(with_counts):
    outs = [jax.ShapeDtypeStruct((N, F), jnp.float32),
            jax.ShapeDtypeStruct((N, F), jnp.float32)]
    if with_counts:
        outs += [jax.ShapeDtypeStruct((N, F), jnp.float32),
                 jax.ShapeDtypeStruct((N, F), jnp.float32)]
    scratch = [pltpu.VMEM((CH,), jnp.int32),
               pltpu.VMEM((CH,), jnp.int32),
               pltpu.VMEM((CH, F), jnp.float32),
               pltpu.VMEM_SHARED((N, F), jnp.float32),
               pltpu.SemaphoreType.DMA]
    mesh = plsc.VectorSubcoreMesh(core_axis_name="c", subcore_axis_name="s")

    @functools.partial(pl.kernel, out_type=outs, mesh=mesh,
                       scratch_types=scratch)
    def agg_kernel(h0, h1, s0, d0, s1, d1, *rest):
        if with_counts:
            agg0, agg1, cnt0, cnt1, idx_s, idx_d, rows, acc, sem = rest
        else:
            agg0, agg1, idx_s, idx_d, rows, acc, sem = rest
            cnt0 = cnt1 = None
        c = lax.axis_index("c")
        s = lax.axis_index("s")
        nblk = N // CH
        full_rounds = nblk // NS

        def fill_rows(val):
            def zr(r, carry):
                for k in range(F // L):
                    rows[r, pl.ds(k * L, L)] = jnp.full((L,), val,
                                                        jnp.float32)
                return carry
            lax.fori_loop(0, CH, zr, 0)

        def zero_acc():
            for bi in range(full_rounds):
                pltpu.sync_copy(rows, acc.at[pl.ds((s + bi * NS) * CH, CH)])

            @pl.when(s < nblk - full_rounds * NS)
            def _():
                pltpu.sync_copy(
                    rows, acc.at[pl.ds((s + full_rounds * NS) * CH, CH)])

        def write_acc(out):
            for bi in range(full_rounds):
                r0 = (s + bi * NS) * CH
                pltpu.sync_copy(acc.at[pl.ds(r0, CH)], out.at[pl.ds(r0, CH)])

            @pl.when(s < nblk - full_rounds * NS)
            def _():
                r0 = (s + full_rounds * NS) * CH
                pltpu.sync_copy(acc.at[pl.ds(r0, CH)], out.at[pl.ds(r0, CH)])

        def run(h, se, de, aggo, cnto):
            fill_rows(0.0)
            zero_acc()
            plsc.subcore_barrier()

            e0 = s * EPT

            def step(j, carry):
                base = e0 + j * CH
                pltpu.sync_copy(se.at[pl.ds(base, CH)], idx_s)
                pltpu.sync_copy(de.at[pl.ds(base, CH)], idx_d)
                pltpu.async_copy(h.at[idx_s], rows, sem).wait()
                pltpu.sync_copy(rows, acc.at[idx_d], add=True)
                return carry
            lax.fori_loop(0, NCHUNK, step, 0)
            plsc.subcore_barrier()
            write_acc(aggo)
            if with_counts:
                plsc.subcore_barrier()
                fill_rows(0.0)
                zero_acc()
                fill_rows(1.0)
                plsc.subcore_barrier()

                def cstep(j, carry):
                    base = e0 + j * CH
                    pltpu.sync_copy(de.at[pl.ds(base, CH)], idx_d)
                    pltpu.sync_copy(rows, acc.at[idx_d], add=True)
                    return carry
                lax.fori_loop(0, NCHUNK, cstep, 0)
                plsc.subcore_barrier()
                write_acc(cnto)

        @pl.when(c == 0)
        def _():
            run(h0, s0, d0, agg0, cnt0)

        @pl.when(c == 1)
        def _():
            run(h1, s1, d1, agg1, cnt1)

    return agg_kernel


_agg_l1 = _make_agg(True)
_agg_l2 = _make_agg(False)

_CONTRACT_T = (((1,), (1,)), ((), ()))


def _proj(x, w, scale, shift):
    BM = 1000

    def body(x_ref, w_ref, sc_ref, sh_ref, o_ref):
        acc = lax.dot_general(x_ref[...], w_ref[...], _CONTRACT_T,
                              preferred_element_type=jnp.float32)
        o_ref[...] = jnp.maximum(acc * sc_ref[...] + sh_ref[...], 0.0)

    return pl.pallas_call(
        body,
        grid=(N // BM,),
        in_specs=[pl.BlockSpec((BM, F), lambda i: (i, 0)),
                  pl.BlockSpec((F, F), lambda i: (0, 0)),
                  pl.BlockSpec((1, F), lambda i: (0, 0)),
                  pl.BlockSpec((1, F), lambda i: (0, 0))],
        out_specs=pl.BlockSpec((BM, F), lambda i: (i, 0)),
        out_shape=jax.ShapeDtypeStruct((N, F), jnp.float32),
    )(x, w, scale, shift)


def _recip(cnt0, cnt1):
    def body(c0_ref, c1_ref, r0_ref, r1_ref):
        for cr, rr in ((c0_ref, r0_ref), (c1_ref, r1_ref)):
            rr[...] = 1.0 / jnp.maximum(cr[:, 0:1], 1.0)

    return pl.pallas_call(
        body,
        out_shape=[jax.ShapeDtypeStruct((N, 1), jnp.float32),
                   jax.ShapeDtypeStruct((N, 1), jnp.float32)],
    )(cnt0, cnt1)


def _combine(aggs, recip, h, wl, wr, b, relu):
    BM = 1000
    aw = aggs.shape[1]
    hw = h.shape[1]

    def body(a_ref, r_ref, h_ref, wl_ref, wr_ref, b_ref, o_ref):
        a = a_ref[...] * r_ref[...]
        out = (lax.dot_general(a, wl_ref[...], _CONTRACT_T,
                               preferred_element_type=jnp.float32)
               + lax.dot_general(h_ref[...], wr_ref[...], _CONTRACT_T,
                                 preferred_element_type=jnp.float32)
               + b_ref[...])
        o_ref[...] = jnp.maximum(out, 0.0) if relu else out

    return pl.pallas_call(
        body,
        grid=(N // BM,),
        in_specs=[pl.BlockSpec((BM, aw), lambda i: (i, 0)),
                  pl.BlockSpec((BM, 1), lambda i: (i, 0)),
                  pl.BlockSpec((BM, hw), lambda i: (i, 0)),
                  pl.BlockSpec((F, F), lambda i: (0, 0)),
                  pl.BlockSpec((F, F), lambda i: (0, 0)),
                  pl.BlockSpec((1, F), lambda i: (0, 0))],
        out_specs=pl.BlockSpec((BM, F), lambda i: (i, 0)),
        out_shape=jax.ShapeDtypeStruct((N, F), jnp.float32),
    )(aggs, recip, h, wl, wr, b)


def kernel(x_user, x_movie, edge_index_rates, edge_index_rated_by,
           lin_user_W, lin_user_b, lin_movie_W, lin_movie_b,
           bn_user_g, bn_user_beta, bn_user_m, bn_user_v,
           bn_movie_g, bn_movie_beta, bn_movie_m, bn_movie_v,
           c1_rates_Wl, c1_rates_bl, c1_rates_Wr,
           c1_rb_Wl, c1_rb_bl, c1_rb_Wr,
           c2_rates_Wl, c2_rates_bl, c2_rates_Wr,
           c2_rb_Wl, c2_rb_bl, c2_rb_Wr):
    eps = 1e-5
    su = edge_index_rates[0].astype(jnp.int32)
    dm = edge_index_rates[1].astype(jnp.int32)
    sm = edge_index_rated_by[0].astype(jnp.int32)
    du = edge_index_rated_by[1].astype(jnp.int32)

    scl_u = bn_user_g / jnp.sqrt(bn_user_v + eps)
    sh_u = (lin_user_b - bn_user_m) * scl_u + bn_user_beta
    scl_m = bn_movie_g / jnp.sqrt(bn_movie_v + eps)
    sh_m = (lin_movie_b - bn_movie_m) * scl_m + bn_movie_beta

    hu = _proj(x_user, lin_user_W, scl_u[None, :], sh_u[None, :])
    hm = _proj(x_movie, lin_movie_W, scl_m[None, :], sh_m[None, :])

    aggm, aggu, cm, cu = _agg_l1(hu, hm, su, dm, sm, du)
    rm, ru = _recip(cm, cu)

    m1 = _combine(aggm, rm, hm, c1_rates_Wl, c1_rates_Wr,
                  c1_rates_bl[None, :], True)
    u1 = _combine(aggu, ru, hu, c1_rb_Wl, c1_rb_Wr,
                  c1_rb_bl[None, :], True)

    aggm2, aggu2 = _agg_l2(u1, m1, su, dm, sm, du)

    m2 = _combine(aggm2, rm, m1, c2_rates_Wl, c2_rates_Wr,
                  c2_rates_bl[None, :], False)
    u2 = _combine(aggu2, ru, u1, c2_rb_Wl, c2_rb_Wr,
                  c2_rb_bl[None, :], False)
    return (u2, m2)


# speedup vs baseline: 5.1579x; 1.6507x over previous
import functools

import jax
import jax.numpy as jnp
from jax import lax
from jax.experimental import pallas as pl
from jax.experimental.pallas import tpu as pltpu
from jax.experimental.pallas import tpu_sc as plsc

N = 10000
E = 320000
F = 128
L = 16
NS = 16
EPT = E // NS
CH = 80
NCHUNK = EPT // CH


def _make_agg(with_counts):
    outs = [jax.ShapeDtypeStruct((N, F), jnp.float32),
            jax.ShapeDtypeStruct((N, F), jnp.float32)]
    if with_counts:
        outs += [jax.ShapeDtypeStruct((N, F), jnp.float32),
                 jax.ShapeDtypeStruct((N, F), jnp.float32)]
    scratch = [pltpu.VMEM((CH,), jnp.int32),
               pltpu.VMEM((CH,), jnp.int32),
               pltpu.VMEM((CH,), jnp.int32),
               pltpu.VMEM((CH,), jnp.int32),
               pltpu.VMEM((CH, F), jnp.float32),
               pltpu.VMEM((CH, F), jnp.float32),
               pltpu.VMEM_SHARED((N, F), jnp.float32),
               pltpu.SemaphoreType.DMA]
    mesh = plsc.VectorSubcoreMesh(core_axis_name="c", subcore_axis_name="s")

    @functools.partial(pl.kernel, out_type=outs, mesh=mesh,
                       scratch_types=scratch)
    def agg_kernel(h0, h1, s0, d0, s1, d1, *rest):
        if with_counts:
            (agg0, agg1, cnt0, cnt1, ixs0, ixs1, ixd0, ixd1, r0, r1, acc,
             sem) = rest
        else:
            agg0, agg1, ixs0, ixs1, ixd0, ixd1, r0, r1, acc, sem = rest
            cnt0 = cnt1 = None
        ixs = (ixs0, ixs1)
        ixd = (ixd0, ixd1)
        rows = (r0, r1)
        c = lax.axis_index("c")
        s = lax.axis_index("s")
        nblk = N // CH
        full_rounds = nblk // NS

        def fill0(val):
            def zr(r, carry):
                for k in range(F // L):
                    rows[0][r, pl.ds(k * L, L)] = jnp.full((L,), val,
                                                           jnp.float32)
                return carry
            lax.fori_loop(0, CH, zr, 0)

        def zero_acc():
            for bi in range(full_rounds):
                pltpu.sync_copy(rows[0],
                                acc.at[pl.ds((s + bi * NS) * CH, CH)])

            @pl.when(s < nblk - full_rounds * NS)
            def _():
                pltpu.sync_copy(
                    rows[0], acc.at[pl.ds((s + full_rounds * NS) * CH, CH)])

        def write_acc(out):
            for bi in range(full_rounds):
                b0 = (s + bi * NS) * CH
                pltpu.sync_copy(acc.at[pl.ds(b0, CH)], out.at[pl.ds(b0, CH)])

            @pl.when(s < nblk - full_rounds * NS)
            def _():
                b0 = (s + full_rounds * NS) * CH
                pltpu.sync_copy(acc.at[pl.ds(b0, CH)], out.at[pl.ds(b0, CH)])

        def run(h, se, de, aggo, cnto):
            fill0(0.0)
            zero_acc()
            plsc.subcore_barrier()

            e0 = s * EPT

            def load_pair(j, slot):
                pltpu.sync_copy(se.at[pl.ds(e0 + j * CH, CH)], ixs[slot])
                pltpu.sync_copy(de.at[pl.ds(e0 + j * CH, CH)], ixd[slot])

            def gather_start(slot):
                pltpu.async_copy(h.at[ixs[slot]], rows[slot], sem)

            def gather_wait(slot):
                pltpu.make_async_copy(h.at[ixs[slot]], rows[slot],
                                      sem).wait()

            def scatter(slot):
                pltpu.sync_copy(rows[slot], acc.at[ixd[slot]], add=True)

            load_pair(0, 0)
            load_pair(1, 1)
            gather_start(0)

            def body(k, carry):
                j = 2 * k
                gather_wait(0)
                gather_start(1)
                scatter(0)
                load_pair(j + 2, 0)
                gather_wait(1)
                gather_start(0)
                scatter(1)
                load_pair(j + 3, 1)
                return carry
            lax.fori_loop(0, NCHUNK // 2 - 1, body, 0)
            gather_wait(0)
            gather_start(1)
            scatter(0)
            gather_wait(1)
            scatter(1)
            plsc.subcore_barrier()
            write_acc(aggo)
            if with_counts:
                fill0(0.0)
                zero_acc()
                fill0(1.0)
                plsc.subcore_barrier()

                def cload_start(j, slot):
                    pltpu.async_copy(de.at[pl.ds(e0 + j * CH, CH)],
                                     ixd[slot], sem)

                def cload_wait(j, slot):
                    pltpu.make_async_copy(de.at[pl.ds(e0 + j * CH, CH)],
                                          ixd[slot], sem).wait()

                def scatter_ones(slot):
                    pltpu.sync_copy(rows[0], acc.at[ixd[slot]], add=True)

                pltpu.sync_copy(de.at[pl.ds(e0, CH)], ixd[0])

                def cbody(k, carry):
                    j = 2 * k
                    cload_start(j + 1, 1)
                    scatter_ones(0)
                    cload_wait(j + 1, 1)
                    cload_start(j + 2, 0)
                    scatter_ones(1)
                    cload_wait(j + 2, 0)
                    return carry
                lax.fori_loop(0, NCHUNK // 2 - 1, cbody, 0)
                cload_start(NCHUNK - 1, 1)
                scatter_ones(0)
                cload_wait(NCHUNK - 1, 1)
                scatter_ones(1)
                plsc.subcore_barrier()
                write_acc(cnto)

        @pl.when(c == 0)
        def _():
            run(h0, s0, d0, agg0, cnt0)

        @pl.when(c == 1)
        def _():
            run(h1, s1, d1, agg1, cnt1)

    return agg_kernel


_agg_l1 = _make_agg(True)
_agg_l2 = _make_agg(False)

_CONTRACT_T = (((1,), (1,)), ((), ()))


def _proj(x, w, scale, shift):
    BM = 1000

    def body(x_ref, w_ref, sc_ref, sh_ref, o_ref):
        acc = lax.dot_general(x_ref[...], w_ref[...], _CONTRACT_T,
                              preferred_element_type=jnp.float32)
        o_ref[...] = jnp.maximum(acc * sc_ref[...] + sh_ref[...], 0.0)

    return pl.pallas_call(
        body,
        grid=(N // BM,),
        in_specs=[pl.BlockSpec((BM, F), lambda i: (i, 0)),
                  pl.BlockSpec((F, F), lambda i: (0, 0)),
                  pl.BlockSpec((1, F), lambda i: (0, 0)),
                  pl.BlockSpec((1, F), lambda i: (0, 0))],
        out_specs=pl.BlockSpec((BM, F), lambda i: (i, 0)),
        out_shape=jax.ShapeDtypeStruct((N, F), jnp.float32),
    )(x, w, scale, shift)


def _recip(cnt0, cnt1):
    def body(c0_ref, c1_ref, r0_ref, r1_ref):
        for cr, rr in ((c0_ref, r0_ref), (c1_ref, r1_ref)):
            rr[...] = 1.0 / jnp.maximum(cr[:, 0:1], 1.0)

    return pl.pallas_call(
        body,
        out_shape=[jax.ShapeDtypeStruct((N, 1), jnp.float32),
                   jax.ShapeDtypeStruct((N, 1), jnp.float32)],
    )(cnt0, cnt1)


def _combine(aggs, recip, h, wl, wr, b, relu):
    BM = 1000
    aw = aggs.shape[1]
    hw = h.shape[1]

    def body(a_ref, r_ref, h_ref, wl_ref, wr_ref, b_ref, o_ref):
        a = a_ref[...] * r_ref[...]
        out = (lax.dot_general(a, wl_ref[...], _CONTRACT_T,
                               preferred_element_type=jnp.float32)
               + lax.dot_general(h_ref[...], wr_ref[...], _CONTRACT_T,
                                 preferred_element_type=jnp.float32)
               + b_ref[...])
        o_ref[...] = jnp.maximum(out, 0.0) if relu else out

    return pl.pallas_call(
        body,
        grid=(N // BM,),
        in_specs=[pl.BlockSpec((BM, aw), lambda i: (i, 0)),
                  pl.BlockSpec((BM, 1), lambda i: (i, 0)),
                  pl.BlockSpec((BM, hw), lambda i: (i, 0)),
                  pl.BlockSpec((F, F), lambda i: (0, 0)),
                  pl.BlockSpec((F, F), lambda i: (0, 0)),
                  pl.BlockSpec((1, F), lambda i: (0, 0))],
        out_specs=pl.BlockSpec((BM, F), lambda i: (i, 0)),
        out_shape=jax.ShapeDtypeStruct((N, F), jnp.float32),
    )(aggs, recip, h, wl, wr, b)


def kernel(x_user, x_movie, edge_index_rates, edge_index_rated_by,
           lin_user_W, lin_user_b, lin_movie_W, lin_movie_b,
           bn_user_g, bn_user_beta, bn_user_m, bn_user_v,
           bn_movie_g, bn_movie_beta, bn_movie_m, bn_movie_v,
           c1_rates_Wl, c1_rates_bl, c1_rates_Wr,
           c1_rb_Wl, c1_rb_bl, c1_rb_Wr,
           c2_rates_Wl, c2_rates_bl, c2_rates_Wr,
           c2_rb_Wl, c2_rb_bl, c2_rb_Wr):
    eps = 1e-5
    su = edge_index_rates[0].astype(jnp.int32)
    dm = edge_index_rates[1].astype(jnp.int32)
    sm = edge_index_rated_by[0].astype(jnp.int32)
    du = edge_index_rated_by[1].astype(jnp.int32)

    scl_u = bn_user_g / jnp.sqrt(bn_user_v + eps)
    sh_u = (lin_user_b - bn_user_m) * scl_u + bn_user_beta
    scl_m = bn_movie_g / jnp.sqrt(bn_movie_v + eps)
    sh_m = (lin_movie_b - bn_movie_m) * scl_m + bn_movie_beta

    hu = _proj(x_user, lin_user_W, scl_u[None, :], sh_u[None, :])
    hm = _proj(x_movie, lin_movie_W, scl_m[None, :], sh_m[None, :])

    aggm, aggu, cm, cu = _agg_l1(hu, hm, su, dm, sm, du)
    rm, ru = _recip(cm, cu)

    m1 = _combine(aggm, rm, hm, c1_rates_Wl, c1_rates_Wr,
                  c1_rates_bl[None, :], True)
    u1 = _combine(aggu, ru, hu, c1_rb_Wl, c1_rb_Wr,
                  c1_rb_bl[None, :], True)

    aggm2, aggu2 = _agg_l2(u1, m1, su, dm, sm, du)

    m2 = _combine(aggm2, rm, m1, c2_rates_Wl, c2_rates_Wr,
                  c2_rates_bl[None, :], False)
    u2 = _combine(aggu2, ru, u1, c2_rb_Wl, c2_rb_Wr,
                  c2_rb_bl[None, :], False)
    return (u2, m2)


# speedup vs baseline: 5.3252x; 1.0324x over previous
import functools

import jax
import jax.numpy as jnp
from jax import lax
from jax.experimental import pallas as pl
from jax.experimental.pallas import tpu as pltpu
from jax.experimental.pallas import tpu_sc as plsc

N = 10000
E = 320000
F = 128
L = 16
NS = 16
EPT = E // NS
CH = 80
NCHUNK = EPT // CH


def _make_agg(with_counts):
    outs = [jax.ShapeDtypeStruct((N, F), jnp.float32),
            jax.ShapeDtypeStruct((N, F), jnp.float32)]
    if with_counts:
        outs += [jax.ShapeDtypeStruct((N, F), jnp.float32),
                 jax.ShapeDtypeStruct((N, F), jnp.float32)]
    scratch = [pltpu.VMEM((CH,), jnp.int32),
               pltpu.VMEM((CH,), jnp.int32),
               pltpu.VMEM((CH,), jnp.int32),
               pltpu.VMEM((CH,), jnp.int32),
               pltpu.VMEM((CH, F), jnp.float32),
               pltpu.VMEM((CH, F), jnp.float32),
               pltpu.VMEM_SHARED((N, F), jnp.float32),
               pltpu.SemaphoreType.DMA]
    mesh = plsc.VectorSubcoreMesh(core_axis_name="c", subcore_axis_name="s")

    @functools.partial(pl.kernel, out_type=outs, mesh=mesh,
                       scratch_types=scratch)
    def agg_kernel(h0, h1, s0, d0, s1, d1, *rest):
        if with_counts:
            (agg0, agg1, cnt0, cnt1, ixs0, ixs1, ixd0, ixd1, r0, r1, acc,
             sem) = rest
        else:
            agg0, agg1, ixs0, ixs1, ixd0, ixd1, r0, r1, acc, sem = rest
            cnt0 = cnt1 = None
        ixs = (ixs0, ixs1)
        ixd = (ixd0, ixd1)
        rows = (r0, r1)
        c = lax.axis_index("c")
        s = lax.axis_index("s")
        nblk = N // CH
        full_rounds = nblk // NS

        def fill0(val):
            def zr(r, carry):
                for k in range(F // L):
                    rows[0][r, pl.ds(k * L, L)] = jnp.full((L,), val,
                                                           jnp.float32)
                return carry
            lax.fori_loop(0, CH, zr, 0)

        def zero_acc():
            for bi in range(full_rounds):
                pltpu.sync_copy(rows[0],
                                acc.at[pl.ds((s + bi * NS) * CH, CH)])

            @pl.when(s < nblk - full_rounds * NS)
            def _():
                pltpu.sync_copy(
                    rows[0], acc.at[pl.ds((s + full_rounds * NS) * CH, CH)])

        def write_acc(out):
            for bi in range(full_rounds):
                b0 = (s + bi * NS) * CH
                pltpu.sync_copy(acc.at[pl.ds(b0, CH)], out.at[pl.ds(b0, CH)])

            @pl.when(s < nblk - full_rounds * NS)
            def _():
                b0 = (s + full_rounds * NS) * CH
                pltpu.sync_copy(acc.at[pl.ds(b0, CH)], out.at[pl.ds(b0, CH)])

        def run(h, se, de, aggo, cnto):
            fill0(0.0)
            zero_acc()
            plsc.subcore_barrier()

            e0 = s * EPT

            def load_pair(j, slot):
                pltpu.sync_copy(se.at[pl.ds(e0 + j * CH, CH)], ixs[slot])
                pltpu.sync_copy(de.at[pl.ds(e0 + j * CH, CH)], ixd[slot])

            def gather_start(slot):
                pltpu.async_copy(h.at[ixs[slot]], rows[slot], sem)

            def gather_wait(slot):
                pltpu.make_async_copy(h.at[ixs[slot]], rows[slot],
                                      sem).wait()

            def scatter(slot):
                pltpu.sync_copy(rows[slot], acc.at[ixd[slot]], add=True)

            load_pair(0, 0)
            load_pair(1, 1)
            gather_start(0)

            def body(k, carry):
                j = 2 * k
                gather_wait(0)
                gather_start(1)
                scatter(0)
                load_pair(j + 2, 0)
                gather_wait(1)
                gather_start(0)
                scatter(1)
                load_pair(j + 3, 1)
                return carry
            lax.fori_loop(0, NCHUNK // 2 - 1, body, 0)
            gather_wait(0)
            gather_start(1)
            scatter(0)
            gather_wait(1)
            scatter(1)
            plsc.subcore_barrier()
            write_acc(aggo)
            if with_counts:
                fill0(0.0)
                zero_acc()
                fill0(1.0)
                plsc.subcore_barrier()

                def cload_start(j, slot):
                    pltpu.async_copy(de.at[pl.ds(e0 + j * CH, CH)],
                                     ixd[slot], sem)

                def cload_wait(j, slot):
                    pltpu.make_async_copy(de.at[pl.ds(e0 + j * CH, CH)],
                                          ixd[slot], sem).wait()

                def scatter_ones(slot):
                    pltpu.sync_copy(rows[0], acc.at[ixd[slot]], add=True)

                pltpu.sync_copy(de.at[pl.ds(e0, CH)], ixd[0])

                def cbody(k, carry):
                    j = 2 * k
                    cload_start(j + 1, 1)
                    scatter_ones(0)
                    cload_wait(j + 1, 1)
                    cload_start(j + 2, 0)
                    scatter_ones(1)
                    cload_wait(j + 2, 0)
                    return carry
                lax.fori_loop(0, NCHUNK // 2 - 1, cbody, 0)
                cload_start(NCHUNK - 1, 1)
                scatter_ones(0)
                cload_wait(NCHUNK - 1, 1)
                scatter_ones(1)
                plsc.subcore_barrier()
                write_acc(cnto)

        @pl.when(c == 0)
        def _():
            run(h0, s0, d0, agg0, cnt0)

        @pl.when(c == 1)
        def _():
            run(h1, s1, d1, agg1, cnt1)

    return agg_kernel


_agg_l1 = _make_agg(True)
_agg_l2 = _make_agg(False)

_CONTRACT_T = (((1,), (1,)), ((), ()))
_BM = 1000
_ROWS = pl.BlockSpec((_BM, F), lambda i: (i, 0))
_WMAT = pl.BlockSpec((F, F), lambda i: (0, 0))
_VEC = pl.BlockSpec((1, F), lambda i: (0, 0))


def _proj2(xu, wu, su_, bu_, xm, wm, sm_, bm_):
    def body(xu_ref, wu_ref, su_ref, bu_ref, xm_ref, wm_ref, sm_ref, bm_ref,
             ou_ref, om_ref):
        for x_ref, w_ref, sc_ref, sh_ref, o_ref in (
                (xu_ref, wu_ref, su_ref, bu_ref, ou_ref),
                (xm_ref, wm_ref, sm_ref, bm_ref, om_ref)):
            acc = lax.dot_general(x_ref[...], w_ref[...], _CONTRACT_T,
                                  preferred_element_type=jnp.float32)
            o_ref[...] = jnp.maximum(acc * sc_ref[...] + sh_ref[...], 0.0)

    return pl.pallas_call(
        body,
        grid=(N // _BM,),
        in_specs=[_ROWS, _WMAT, _VEC, _VEC, _ROWS, _WMAT, _VEC, _VEC],
        out_specs=[_ROWS, _ROWS],
        out_shape=[jax.ShapeDtypeStruct((N, F), jnp.float32),
                   jax.ShapeDtypeStruct((N, F), jnp.float32)],
    )(xu, wu, su_, bu_, xm, wm, sm_, bm_)


def _combine2(aggm, cntm, hm, wlm, wrm, bm_, aggu, cntu, hu, wlu, wru, bu_,
              relu):
    def one(a_ref, c_ref, h_ref, wl_ref, wr_ref, b_ref, o_ref):
        r = 1.0 / jnp.maximum(c_ref[:, 0:1], 1.0)
        out = (lax.dot_general(a_ref[...] * r, wl_ref[...], _CONTRACT_T,
                               preferred_element_type=jnp.float32)
               + lax.dot_general(h_ref[...], wr_ref[...], _CONTRACT_T,
                                 preferred_element_type=jnp.float32)
               + b_ref[...])
        o_ref[...] = jnp.maximum(out, 0.0) if relu else out

    def body(am_ref, cm_ref, hm_ref, wlm_ref, wrm_ref, bm_ref,
             au_ref, cu_ref, hu_ref, wlu_ref, wru_ref, bu_ref,
             om_ref, ou_ref):
        one(am_ref, cm_ref, hm_ref, wlm_ref, wrm_ref, bm_ref, om_ref)
        one(au_ref, cu_ref, hu_ref, wlu_ref, wru_ref, bu_ref, ou_ref)

    return pl.pallas_call(
        body,
        grid=(N // _BM,),
        in_specs=[_ROWS, _ROWS, _ROWS, _WMAT, _WMAT, _VEC,
                  _ROWS, _ROWS, _ROWS, _WMAT, _WMAT, _VEC],
        out_specs=[_ROWS, _ROWS],
        out_shape=[jax.ShapeDtypeStruct((N, F), jnp.float32),
                   jax.ShapeDtypeStruct((N, F), jnp.float32)],
    )(aggm, cntm, hm, wlm, wrm, bm_, aggu, cntu, hu, wlu, wru, bu_)


def kernel(x_user, x_movie, edge_index_rates, edge_index_rated_by,
           lin_user_W, lin_user_b, lin_movie_W, lin_movie_b,
           bn_user_g, bn_user_beta, bn_user_m, bn_user_v,
           bn_movie_g, bn_movie_beta, bn_movie_m, bn_movie_v,
           c1_rates_Wl, c1_rates_bl, c1_rates_Wr,
           c1_rb_Wl, c1_rb_bl, c1_rb_Wr,
           c2_rates_Wl, c2_rates_bl, c2_rates_Wr,
           c2_rb_Wl, c2_rb_bl, c2_rb_Wr):
    eps = 1e-5
    su = edge_index_rates[0].astype(jnp.int32)
    dm = edge_index_rates[1].astype(jnp.int32)
    sm = edge_index_rated_by[0].astype(jnp.int32)
    du = edge_index_rated_by[1].astype(jnp.int32)

    scl_u = bn_user_g / jnp.sqrt(bn_user_v + eps)
    sh_u = (lin_user_b - bn_user_m) * scl_u + bn_user_beta
    scl_m = bn_movie_g / jnp.sqrt(bn_movie_v + eps)
    sh_m = (lin_movie_b - bn_movie_m) * scl_m + bn_movie_beta

    hu, hm = _proj2(x_user, lin_user_W, scl_u[None, :], sh_u[None, :],
                    x_movie, lin_movie_W, scl_m[None, :], sh_m[None, :])

    aggm, aggu, cm, cu = _agg_l1(hu, hm, su, dm, sm, du)

    m1, u1 = _combine2(aggm, cm, hm, c1_rates_Wl, c1_rates_Wr,
                       c1_rates_bl[None, :],
                       aggu, cu, hu, c1_rb_Wl, c1_rb_Wr,
                       c1_rb_bl[None, :], True)

    aggm2, aggu2 = _agg_l2(u1, m1, su, dm, sm, du)

    m2, u2 = _combine2(aggm2, cm, m1, c2_rates_Wl, c2_rates_Wr,
                       c2_rates_bl[None, :],
                       aggu2, cu, u1, c2_rb_Wl, c2_rb_Wr,
                       c2_rb_bl[None, :], False)
    return (u2, m2)


# speedup vs baseline: 6.0705x; 1.1400x over previous
import functools

import jax
import jax.numpy as jnp
from jax import lax
from jax.experimental import pallas as pl
from jax.experimental.pallas import tpu as pltpu
from jax.experimental.pallas import tpu_sc as plsc

N = 10000
E = 320000
F = 128
L = 16
NS = 16
EPT = E // NS
CH = 80
NCHUNK = EPT // CH


def _make_agg(with_counts):
    outs = [jax.ShapeDtypeStruct((N, F), jnp.float32),
            jax.ShapeDtypeStruct((N, F), jnp.float32)]
    if with_counts:
        outs += [jax.ShapeDtypeStruct((N, F), jnp.float32),
                 jax.ShapeDtypeStruct((N, F), jnp.float32)]
    scratch = ([pltpu.VMEM((CH,), jnp.int32)] * 6
               + [pltpu.VMEM((CH, F), jnp.float32)] * 2
               + [pltpu.VMEM_SHARED((N, F), jnp.float32),
                  pltpu.SemaphoreType.DMA, pltpu.SemaphoreType.DMA])
    mesh = plsc.VectorSubcoreMesh(core_axis_name="c", subcore_axis_name="s")

    @functools.partial(pl.kernel, out_type=outs, mesh=mesh,
                       scratch_types=scratch)
    def agg_kernel(h0, h1, s0, d0, s1, d1, *rest):
        if with_counts:
            (agg0, agg1, cnt0, cnt1, ixs0, ixs1, ixd0, ixd1, ixd2, ixd3,
             r0, r1, acc, sem, ssem) = rest
        else:
            (agg0, agg1, ixs0, ixs1, ixd0, ixd1, ixd2, ixd3, r0, r1, acc,
             sem, ssem) = rest
            cnt0 = cnt1 = None
        ixs = (ixs0, ixs1)
        ixd = (ixd0, ixd1, ixd2, ixd3)
        rows = (r0, r1)
        c = lax.axis_index("c")
        s = lax.axis_index("s")
        nblk = N // CH
        full_rounds = nblk // NS

        def fill0(val):
            def zr(r, carry):
                for k in range(F // L):
                    rows[0][r, pl.ds(k * L, L)] = jnp.full((L,), val,
                                                           jnp.float32)
                return carry
            lax.fori_loop(0, CH, zr, 0)

        def zero_acc():
            for bi in range(full_rounds):
                pltpu.sync_copy(rows[0],
                                acc.at[pl.ds((s + bi * NS) * CH, CH)])

            @pl.when(s < nblk - full_rounds * NS)
            def _():
                pltpu.sync_copy(
                    rows[0], acc.at[pl.ds((s + full_rounds * NS) * CH, CH)])

        def write_acc(out):
            for bi in range(full_rounds):
                b0 = (s + bi * NS) * CH
                pltpu.sync_copy(acc.at[pl.ds(b0, CH)], out.at[pl.ds(b0, CH)])

            @pl.when(s < nblk - full_rounds * NS)
            def _():
                b0 = (s + full_rounds * NS) * CH
                pltpu.sync_copy(acc.at[pl.ds(b0, CH)], out.at[pl.ds(b0, CH)])

        def run(h, se, de, aggo, cnto):
            fill0(0.0)
            zero_acc()
            plsc.subcore_barrier()

            e0 = s * EPT

            def load_pair(j, s_slot, d_slot):
                pltpu.sync_copy(se.at[pl.ds(e0 + j * CH, CH)], ixs[s_slot])
                pltpu.sync_copy(de.at[pl.ds(e0 + j * CH, CH)], ixd[d_slot])

            def gather_start(slot):
                pltpu.async_copy(h.at[ixs[slot]], rows[slot], sem)

            def gather_wait(slot):
                pltpu.make_async_copy(h.at[ixs[slot]], rows[slot],
                                      sem).wait()

            def scatter_start(a_slot, d_slot):
                pltpu.async_copy(rows[a_slot], acc.at[ixd[d_slot]], ssem,
                                 add=True)

            def scatter_wait():
                pltpu.make_async_copy(rows[0], acc.at[ixd[0]], ssem).wait()

            def sub(jv, i, first=False, gather_next=True, load_next=True):
                a = i % 2
                gather_wait(a)
                scatter_start(a, i % 4)
                if not first:
                    scatter_wait()
                if gather_next:
                    gather_start(1 - a)
                if load_next:
                    load_pair(jv + 2, a, (i + 2) % 4)

            load_pair(0, 0, 0)
            load_pair(1, 1, 1)
            gather_start(0)
            sub(0, 0, first=True)
            sub(1, 1)
            sub(2, 2)
            sub(3, 3)

            def body(k, carry):
                j0 = 4 * k
                for i in range(4):
                    sub(j0 + i, i)
                return carry
            lax.fori_loop(1, NCHUNK // 4, body, 0)
            sub(NCHUNK - 2, 0, load_next=False)
            sub(NCHUNK - 1, 1, gather_next=False, load_next=False)
            scatter_wait()
            plsc.subcore_barrier()
            write_acc(aggo)
            if with_counts:
                fill0(0.0)
                zero_acc()
                fill0(1.0)
                plsc.subcore_barrier()

                def cload_start(j, slot):
                    pltpu.async_copy(de.at[pl.ds(e0 + j * CH, CH)],
                                     ixd[slot], sem)

                def cload_wait(j, slot):
                    pltpu.make_async_copy(de.at[pl.ds(e0 + j * CH, CH)],
                                          ixd[slot], sem).wait()

                def scatter_ones(slot):
                    pltpu.sync_copy(rows[0], acc.at[ixd[slot]], add=True)

                pltpu.sync_copy(de.at[pl.ds(e0, CH)], ixd[0])

                def cbody(k, carry):
                    j = 2 * k
                    cload_start(j + 1, 1)
                    scatter_ones(0)
                    cload_wait(j + 1, 1)
                    cload_start(j + 2, 0)
                    scatter_ones(1)
                    cload_wait(j + 2, 0)
                    return carry
                lax.fori_loop(0, NCHUNK // 2 - 1, cbody, 0)
                cload_start(NCHUNK - 1, 1)
                scatter_ones(0)
                cload_wait(NCHUNK - 1, 1)
                scatter_ones(1)
                plsc.subcore_barrier()
                write_acc(cnto)

        @pl.when(c == 0)
        def _():
            run(h0, s0, d0, agg0, cnt0)

        @pl.when(c == 1)
        def _():
            run(h1, s1, d1, agg1, cnt1)

    return agg_kernel


_agg_l1 = _make_agg(True)
_agg_l2 = _make_agg(False)

_CONTRACT_T = (((1,), (1,)), ((), ()))
_BM = 1000
_ROWS = pl.BlockSpec((_BM, F), lambda i: (i, 0))
_WMAT = pl.BlockSpec((F, F), lambda i: (0, 0))
_VEC = pl.BlockSpec((1, F), lambda i: (0, 0))


def _proj2(xu, wu, su_, bu_, xm, wm, sm_, bm_):
    def body(xu_ref, wu_ref, su_ref, bu_ref, xm_ref, wm_ref, sm_ref, bm_ref,
             ou_ref, om_ref):
        for x_ref, w_ref, sc_ref, sh_ref, o_ref in (
                (xu_ref, wu_ref, su_ref, bu_ref, ou_ref),
                (xm_ref, wm_ref, sm_ref, bm_ref, om_ref)):
            acc = lax.dot_general(x_ref[...], w_ref[...], _CONTRACT_T,
                                  preferred_element_type=jnp.float32)
            o_ref[...] = jnp.maximum(acc * sc_ref[...] + sh_ref[...], 0.0)

    return pl.pallas_call(
        body,
        grid=(N // _BM,),
        in_specs=[_ROWS, _WMAT, _VEC, _VEC, _ROWS, _WMAT, _VEC, _VEC],
        out_specs=[_ROWS, _ROWS],
        out_shape=[jax.ShapeDtypeStruct((N, F), jnp.float32),
                   jax.ShapeDtypeStruct((N, F), jnp.float32)],
    )(xu, wu, su_, bu_, xm, wm, sm_, bm_)


def _combine2(aggm, cntm, hm, wlm, wrm, bm_, aggu, cntu, hu, wlu, wru, bu_,
              relu):
    def one(a_ref, c_ref, h_ref, wl_ref, wr_ref, b_ref, o_ref):
        r = 1.0 / jnp.maximum(c_ref[:, 0:1], 1.0)
        out = (lax.dot_general(a_ref[...] * r, wl_ref[...], _CONTRACT_T,
                               preferred_element_type=jnp.float32)
               + lax.dot_general(h_ref[...], wr_ref[...], _CONTRACT_T,
                                 preferred_element_type=jnp.float32)
               + b_ref[...])
        o_ref[...] = jnp.maximum(out, 0.0) if relu else out

    def body(am_ref, cm_ref, hm_ref, wlm_ref, wrm_ref, bm_ref,
             au_ref, cu_ref, hu_ref, wlu_ref, wru_ref, bu_ref,
             om_ref, ou_ref):
        one(am_ref, cm_ref, hm_ref, wlm_ref, wrm_ref, bm_ref, om_ref)
        one(au_ref, cu_ref, hu_ref, wlu_ref, wru_ref, bu_ref, ou_ref)

    return pl.pallas_call(
        body,
        grid=(N // _BM,),
        in_specs=[_ROWS, _ROWS, _ROWS, _WMAT, _WMAT, _VEC,
                  _ROWS, _ROWS, _ROWS, _WMAT, _WMAT, _VEC],
        out_specs=[_ROWS, _ROWS],
        out_shape=[jax.ShapeDtypeStruct((N, F), jnp.float32),
                   jax.ShapeDtypeStruct((N, F), jnp.float32)],
    )(aggm, cntm, hm, wlm, wrm, bm_, aggu, cntu, hu, wlu, wru, bu_)


def kernel(x_user, x_movie, edge_index_rates, edge_index_rated_by,
           lin_user_W, lin_user_b, lin_movie_W, lin_movie_b,
           bn_user_g, bn_user_beta, bn_user_m, bn_user_v,
           bn_movie_g, bn_movie_beta, bn_movie_m, bn_movie_v,
           c1_rates_Wl, c1_rates_bl, c1_rates_Wr,
           c1_rb_Wl, c1_rb_bl, c1_rb_Wr,
           c2_rates_Wl, c2_rates_bl, c2_rates_Wr,
           c2_rb_Wl, c2_rb_bl, c2_rb_Wr):
    eps = 1e-5
    su = edge_index_rates[0].astype(jnp.int32)
    dm = edge_index_rates[1].astype(jnp.int32)
    sm = edge_index_rated_by[0].astype(jnp.int32)
    du = edge_index_rated_by[1].astype(jnp.int32)

    scl_u = bn_user_g / jnp.sqrt(bn_user_v + eps)
    sh_u = (lin_user_b - bn_user_m) * scl_u + bn_user_beta
    scl_m = bn_movie_g / jnp.sqrt(bn_movie_v + eps)
    sh_m = (lin_movie_b - bn_movie_m) * scl_m + bn_movie_beta

    hu, hm = _proj2(x_user, lin_user_W, scl_u[None, :], sh_u[None, :],
                    x_movie, lin_movie_W, scl_m[None, :], sh_m[None, :])

    aggm, aggu, cm, cu = _agg_l1(hu, hm, su, dm, sm, du)

    m1, u1 = _combine2(aggm, cm, hm, c1_rates_Wl, c1_rates_Wr,
                       c1_rates_bl[None, :],
                       aggu, cu, hu, c1_rb_Wl, c1_rb_Wr,
                       c1_rb_bl[None, :], True)

    aggm2, aggu2 = _agg_l2(u1, m1, su, dm, sm, du)

    m2, u2 = _combine2(aggm2, cm, m1, c2_rates_Wl, c2_rates_Wr,
                       c2_rates_bl[None, :],
                       aggu2, cu, u1, c2_rb_Wl, c2_rb_Wr,
                       c2_rb_bl[None, :], False)
    return (u2, m2)


# speedup vs baseline: 6.3279x; 1.0424x over previous
import functools

import jax
import jax.numpy as jnp
from jax import lax
from jax.experimental import pallas as pl
from jax.experimental.pallas import tpu as pltpu
from jax.experimental.pallas import tpu_sc as plsc

N = 10000
E = 320000
F = 128
L = 16
NS = 16
EPT = E // NS
CH = 80
NCHUNK = EPT // CH


def _make_agg(with_counts):
    outs = [jax.ShapeDtypeStruct((N, F), jnp.float32),
            jax.ShapeDtypeStruct((N, F), jnp.float32)]
    if with_counts:
        outs += [jax.ShapeDtypeStruct((N, F), jnp.float32),
                 jax.ShapeDtypeStruct((N, F), jnp.float32)]
    scratch = ([pltpu.VMEM((CH,), jnp.int32)] * 8
               + [pltpu.VMEM((CH, F), jnp.float32)] * 4
               + [pltpu.VMEM_SHARED((N, F), jnp.float32),
                  pltpu.SemaphoreType.DMA, pltpu.SemaphoreType.DMA])
    mesh = plsc.VectorSubcoreMesh(core_axis_name="c", subcore_axis_name="s")

    @functools.partial(pl.kernel, out_type=outs, mesh=mesh,
                       scratch_types=scratch)
    def agg_kernel(h0, h1, s0, d0, s1, d1, *rest):
        if with_counts:
            (agg0, agg1, cnt0, cnt1, ixs0, ixs1, ixs2, ixs3,
             ixd0, ixd1, ixd2, ixd3, r0, r1, r2, r3, acc, sem, ssem) = rest
        else:
            (agg0, agg1, ixs0, ixs1, ixs2, ixs3, ixd0, ixd1, ixd2, ixd3,
             r0, r1, r2, r3, acc, sem, ssem) = rest
            cnt0 = cnt1 = None
        ixs = (ixs0, ixs1, ixs2, ixs3)
        ixd = (ixd0, ixd1, ixd2, ixd3)
        rows = (r0, r1, r2, r3)
        c = lax.axis_index("c")
        s = lax.axis_index("s")
        nblk = N // CH
        full_rounds = nblk // NS

        def fill0(val):
            def zr(r, carry):
                for k in range(F // L):
                    rows[0][r, pl.ds(k * L, L)] = jnp.full((L,), val,
                                                           jnp.float32)
                return carry
            lax.fori_loop(0, CH, zr, 0)

        def zero_acc():
            for bi in range(full_rounds):
                pltpu.sync_copy(rows[0],
                                acc.at[pl.ds((s + bi * NS) * CH, CH)])

            @pl.when(s < nblk - full_rounds * NS)
            def _():
                pltpu.sync_copy(
                    rows[0], acc.at[pl.ds((s + full_rounds * NS) * CH, CH)])

        def write_acc(out):
            for bi in range(full_rounds):
                b0 = (s + bi * NS) * CH
                pltpu.sync_copy(acc.at[pl.ds(b0, CH)], out.at[pl.ds(b0, CH)])

            @pl.when(s < nblk - full_rounds * NS)
            def _():
                b0 = (s + full_rounds * NS) * CH
                pltpu.sync_copy(acc.at[pl.ds(b0, CH)], out.at[pl.ds(b0, CH)])

        def run(h, se, de, aggo, cnto):
            fill0(0.0)
            zero_acc()
            plsc.subcore_barrier()

            e0 = s * EPT

            def load_pair(j, s_slot, d_slot):
                pltpu.sync_copy(se.at[pl.ds(e0 + j * CH, CH)], ixs[s_slot])
                pltpu.sync_copy(de.at[pl.ds(e0 + j * CH, CH)], ixd[d_slot])

            def gather_start(slot):
                pltpu.async_copy(h.at[ixs[slot]], rows[slot], sem)

            def gather_wait(slot):
                pltpu.make_async_copy(h.at[ixs[slot]], rows[slot],
                                      sem).wait()

            def scatter_start(a_slot, d_slot):
                pltpu.async_copy(rows[a_slot], acc.at[ixd[d_slot]], ssem,
                                 add=True)

            def scatter_wait():
                pltpu.make_async_copy(rows[0], acc.at[ixd[0]], ssem).wait()

            def sub(jv, i, first=False, gather_next=True, load_next=True):
                m = i % 4
                gather_wait(m)
                scatter_start(m, m)
                if not first:
                    scatter_wait()
                if load_next:
                    load_pair(jv + 3, (m + 3) % 4, (m + 3) % 4)
                if gather_next:
                    gather_start((m + 2) % 4)

            load_pair(0, 0, 0)
            load_pair(1, 1, 1)
            load_pair(2, 2, 2)
            gather_start(0)
            gather_start(1)
            sub(0, 0, first=True)
            sub(1, 1)
            sub(2, 2)
            sub(3, 3)

            def body(k, carry):
                j0 = 4 * k
                for i in range(4):
                    sub(j0 + i, i)
                return carry
            lax.fori_loop(1, NCHUNK // 4 - 1, body, 0)
            sub(NCHUNK - 6, 0)
            sub(NCHUNK - 5, 1)
            sub(NCHUNK - 4, 2)
            sub(NCHUNK - 3, 3, load_next=False)
            sub(NCHUNK - 2, 0, load_next=False, gather_next=False)
            sub(NCHUNK - 1, 1, load_next=False, gather_next=False)
            scatter_wait()
            plsc.subcore_barrier()
            write_acc(aggo)
            if with_counts:
                fill0(0.0)
                zero_acc()
                fill0(1.0)
                plsc.subcore_barrier()

                def cload_start(j, slot):
                    pltpu.async_copy(de.at[pl.ds(e0 + j * CH, CH)],
                                     ixd[slot], sem)

                def cload_wait(j, slot):
                    pltpu.make_async_copy(de.at[pl.ds(e0 + j * CH, CH)],
                                          ixd[slot], sem).wait()

                def scatter_ones(slot):
                    pltpu.sync_copy(rows[0], acc.at[ixd[slot]], add=True)

                pltpu.sync_copy(de.at[pl.ds(e0, CH)], ixd[0])

                def cbody(k, carry):
                    j = 2 * k
                    cload_start(j + 1, 1)
                    scatter_ones(0)
                    cload_wait(j + 1, 1)
                    cload_start(j + 2, 0)
                    scatter_ones(1)
                    cload_wait(j + 2, 0)
                    return carry
                lax.fori_loop(0, NCHUNK // 2 - 1, cbody, 0)
                cload_start(NCHUNK - 1, 1)
                scatter_ones(0)
                cload_wait(NCHUNK - 1, 1)
                scatter_ones(1)
                plsc.subcore_barrier()
                write_acc(cnto)

        @pl.when(c == 0)
        def _():
            run(h0, s0, d0, agg0, cnt0)

        @pl.when(c == 1)
        def _():
            run(h1, s1, d1, agg1, cnt1)

    return agg_kernel


_agg_l1 = _make_agg(True)
_agg_l2 = _make_agg(False)

_CONTRACT_T = (((1,), (1,)), ((), ()))
_BM = 1000
_ROWS = pl.BlockSpec((_BM, F), lambda i: (i, 0))
_WMAT = pl.BlockSpec((F, F), lambda i: (0, 0))
_VEC = pl.BlockSpec((1, F), lambda i: (0, 0))


def _proj2(xu, wu, su_, bu_, xm, wm, sm_, bm_):
    def body(xu_ref, wu_ref, su_ref, bu_ref, xm_ref, wm_ref, sm_ref, bm_ref,
             ou_ref, om_ref):
        for x_ref, w_ref, sc_ref, sh_ref, o_ref in (
                (xu_ref, wu_ref, su_ref, bu_ref, ou_ref),
                (xm_ref, wm_ref, sm_ref, bm_ref, om_ref)):
            acc = lax.dot_general(x_ref[...], w_ref[...], _CONTRACT_T,
                                  preferred_element_type=jnp.float32)
            o_ref[...] = jnp.maximum(acc * sc_ref[...] + sh_ref[...], 0.0)

    return pl.pallas_call(
        body,
        grid=(N // _BM,),
        in_specs=[_ROWS, _WMAT, _VEC, _VEC, _ROWS, _WMAT, _VEC, _VEC],
        out_specs=[_ROWS, _ROWS],
        out_shape=[jax.ShapeDtypeStruct((N, F), jnp.float32),
                   jax.ShapeDtypeStruct((N, F), jnp.float32)],
    )(xu, wu, su_, bu_, xm, wm, sm_, bm_)


def _combine2(aggm, cntm, hm, wlm, wrm, bm_, aggu, cntu, hu, wlu, wru, bu_,
              relu):
    def one(a_ref, c_ref, h_ref, wl_ref, wr_ref, b_ref, o_ref):
        r = 1.0 / jnp.maximum(c_ref[:, 0:1], 1.0)
        out = (lax.dot_general(a_ref[...] * r, wl_ref[...], _CONTRACT_T,
                               preferred_element_type=jnp.float32)
               + lax.dot_general(h_ref[...], wr_ref[...], _CONTRACT_T,
                                 preferred_element_type=jnp.float32)
               + b_ref[...])
        o_ref[...] = jnp.maximum(out, 0.0) if relu else out

    def body(am_ref, cm_ref, hm_ref, wlm_ref, wrm_ref, bm_ref,
             au_ref, cu_ref, hu_ref, wlu_ref, wru_ref, bu_ref,
             om_ref, ou_ref):
        one(am_ref, cm_ref, hm_ref, wlm_ref, wrm_ref, bm_ref, om_ref)
        one(au_ref, cu_ref, hu_ref, wlu_ref, wru_ref, bu_ref, ou_ref)

    return pl.pallas_call(
        body,
        grid=(N // _BM,),
        in_specs=[_ROWS, _ROWS, _ROWS, _WMAT, _WMAT, _VEC,
                  _ROWS, _ROWS, _ROWS, _WMAT, _WMAT, _VEC],
        out_specs=[_ROWS, _ROWS],
        out_shape=[jax.ShapeDtypeStruct((N, F), jnp.float32),
                   jax.ShapeDtypeStruct((N, F), jnp.float32)],
    )(aggm, cntm, hm, wlm, wrm, bm_, aggu, cntu, hu, wlu, wru, bu_)


def kernel(x_user, x_movie, edge_index_rates, edge_index_rated_by,
           lin_user_W, lin_user_b, lin_movie_W, lin_movie_b,
           bn_user_g, bn_user_beta, bn_user_m, bn_user_v,
           bn_movie_g, bn_movie_beta, bn_movie_m, bn_movie_v,
           c1_rates_Wl, c1_rates_bl, c1_rates_Wr,
           c1_rb_Wl, c1_rb_bl, c1_rb_Wr,
           c2_rates_Wl, c2_rates_bl, c2_rates_Wr,
           c2_rb_Wl, c2_rb_bl, c2_rb_Wr):
    eps = 1e-5
    su = edge_index_rates[0].astype(jnp.int32)
    dm = edge_index_rates[1].astype(jnp.int32)
    sm = edge_index_rated_by[0].astype(jnp.int32)
    du = edge_index_rated_by[1].astype(jnp.int32)

    scl_u = bn_user_g / jnp.sqrt(bn_user_v + eps)
    sh_u = (lin_user_b - bn_user_m) * scl_u + bn_user_beta
    scl_m = bn_movie_g / jnp.sqrt(bn_movie_v + eps)
    sh_m = (lin_movie_b - bn_movie_m) * scl_m + bn_movie_beta

    hu, hm = _proj2(x_user, lin_user_W, scl_u[None, :], sh_u[None, :],
                    x_movie, lin_movie_W, scl_m[None, :], sh_m[None, :])

    aggm, aggu, cm, cu = _agg_l1(hu, hm, su, dm, sm, du)

    m1, u1 = _combine2(aggm, cm, hm, c1_rates_Wl, c1_rates_Wr,
                       c1_rates_bl[None, :],
                       aggu, cu, hu, c1_rb_Wl, c1_rb_Wr,
                       c1_rb_bl[None, :], True)

    aggm2, aggu2 = _agg_l2(u1, m1, su, dm, sm, du)

    m2, u2 = _combine2(aggm2, cm, m1, c2_rates_Wl, c2_rates_Wr,
                       c2_rates_bl[None, :],
                       aggu2, cu, u1, c2_rb_Wl, c2_rb_Wr,
                       c2_rb_bl[None, :], False)
    return (u2, m2)


# speedup vs baseline: 6.7736x; 1.0704x over previous
import functools

import jax
import jax.numpy as jnp
from jax import lax
from jax.experimental import pallas as pl
from jax.experimental.pallas import tpu as pltpu
from jax.experimental.pallas import tpu_sc as plsc

N = 10000
E = 320000
F = 128
L = 16
NS = 16
EPT = E // NS
CH = 80
NCHUNK = EPT // CH


def _make_agg(with_counts):
    outs = [jax.ShapeDtypeStruct((N, F), jnp.float32),
            jax.ShapeDtypeStruct((N, F), jnp.float32)]
    if with_counts:
        outs += [jax.ShapeDtypeStruct((N, F), jnp.float32),
                 jax.ShapeDtypeStruct((N, F), jnp.float32)]
    scratch = ([pltpu.VMEM((CH,), jnp.int32)] * 16
               + [pltpu.VMEM((CH, F), jnp.float32)] * 4
               + [pltpu.VMEM_SHARED((N, F), jnp.float32),
                  pltpu.SemaphoreType.DMA, pltpu.SemaphoreType.DMA])
    mesh = plsc.VectorSubcoreMesh(core_axis_name="c", subcore_axis_name="s")

    @functools.partial(pl.kernel, out_type=outs, mesh=mesh,
                       scratch_types=scratch)
    def agg_kernel(h0, h1, s0, d0, s1, d1, *rest):
        if with_counts:
            agg0, agg1, cnt0, cnt1 = rest[:4]
            rest = rest[4:]
        else:
            agg0, agg1 = rest[:2]
            rest = rest[2:]
            cnt0 = cnt1 = None
        ixs = rest[0:8]
        ixd = rest[8:16]
        rows = rest[16:20]
        acc, sem, ssem = rest[20:23]
        c = lax.axis_index("c")
        s = lax.axis_index("s")
        nblk = N // CH
        full_rounds = nblk // NS

        def fill0(val):
            def zr(r, carry):
                for k in range(F // L):
                    rows[0][r, pl.ds(k * L, L)] = jnp.full((L,), val,
                                                           jnp.float32)
                return carry
            lax.fori_loop(0, CH, zr, 0)

        def zero_acc():
            for bi in range(full_rounds):
                pltpu.sync_copy(rows[0],
                                acc.at[pl.ds((s + bi * NS) * CH, CH)])

            @pl.when(s < nblk - full_rounds * NS)
            def _():
                pltpu.sync_copy(
                    rows[0], acc.at[pl.ds((s + full_rounds * NS) * CH, CH)])

        def write_acc(out):
            for bi in range(full_rounds):
                b0 = (s + bi * NS) * CH
                pltpu.sync_copy(acc.at[pl.ds(b0, CH)], out.at[pl.ds(b0, CH)])

            @pl.when(s < nblk - full_rounds * NS)
            def _():
                b0 = (s + full_rounds * NS) * CH
                pltpu.sync_copy(acc.at[pl.ds(b0, CH)], out.at[pl.ds(b0, CH)])

        def run(h, se, de, aggo, cnto):
            fill0(0.0)
            zero_acc()
            plsc.subcore_barrier()

            e0 = s * EPT

            def load_pair(j, s_slot, d_slot):
                pltpu.sync_copy(se.at[pl.ds(e0 + j * CH, CH)], ixs[s_slot])
                pltpu.sync_copy(de.at[pl.ds(e0 + j * CH, CH)], ixd[d_slot])

            def gather_start(r_slot, s_slot):
                pltpu.async_copy(h.at[ixs[s_slot]], rows[r_slot], sem)

            def gather_wait(r_slot):
                pltpu.make_async_copy(h.at[ixs[0]], rows[r_slot],
                                      sem).wait()

            def scatter_start(a_slot, d_slot):
                pltpu.async_copy(rows[a_slot], acc.at[ixd[d_slot]], ssem,
                                 add=True)

            def scatter_wait():
                pltpu.make_async_copy(rows[0], acc.at[ixd[0]], ssem).wait()

            def sub(jv, im, first=False, gather3=True, load4=True):
                m = im % 4
                gather_wait(m)
                scatter_start(m, im % 8)
                if not first:
                    scatter_wait()
                if load4:
                    load_pair(jv + 4, (im + 4) % 8, (im + 4) % 8)
                if gather3:
                    gather_start((m + 3) % 4, (im + 3) % 8)

            for p in range(4):
                load_pair(p, p, p)
            for p in range(3):
                gather_start(p, p)
            sub(0, 0, first=True)
            for p in range(1, 8):
                sub(p, p)

            def body(k, carry):
                j0 = 8 * k
                for i in range(8):
                    sub(j0 + i, i)
                return carry
            lax.fori_loop(1, (NCHUNK - 10) // 8, body, 0)
            for p in range(NCHUNK - 10, NCHUNK - 4):
                sub(p, p % 8)
            sub(NCHUNK - 4, (NCHUNK - 4) % 8, load4=False)
            sub(NCHUNK - 3, (NCHUNK - 3) % 8, load4=False,
                gather3=False)
            sub(NCHUNK - 2, (NCHUNK - 2) % 8, load4=False,
                gather3=False)
            sub(NCHUNK - 1, (NCHUNK - 1) % 8, load4=False,
                gather3=False)
            scatter_wait()
            plsc.subcore_barrier()
            write_acc(aggo)
            if with_counts:
                fill0(0.0)
                zero_acc()
                fill0(1.0)
                plsc.subcore_barrier()

                def cload_start(j, slot):
                    pltpu.async_copy(de.at[pl.ds(e0 + j * CH, CH)],
                                     ixd[slot], sem)

                def cload_wait(j, slot):
                    pltpu.make_async_copy(de.at[pl.ds(e0 + j * CH, CH)],
                                          ixd[slot], sem).wait()

                def scatter_ones(slot):
                    pltpu.sync_copy(rows[0], acc.at[ixd[slot]], add=True)

                pltpu.sync_copy(de.at[pl.ds(e0, CH)], ixd[0])

                def cbody(k, carry):
                    j = 2 * k
                    cload_start(j + 1, 1)
                    scatter_ones(0)
                    cload_wait(j + 1, 1)
                    cload_start(j + 2, 0)
                    scatter_ones(1)
                    cload_wait(j + 2, 0)
                    return carry
                lax.fori_loop(0, NCHUNK // 2 - 1, cbody, 0)
                cload_start(NCHUNK - 1, 1)
                scatter_ones(0)
                cload_wait(NCHUNK - 1, 1)
                scatter_ones(1)
                plsc.subcore_barrier()
                write_acc(cnto)

        @pl.when(c == 0)
        def _():
            run(h0, s0, d0, agg0, cnt0)

        @pl.when(c == 1)
        def _():
            run(h1, s1, d1, agg1, cnt1)

    return agg_kernel


_agg_l1 = _make_agg(True)
_agg_l2 = _make_agg(False)

_CONTRACT_T = (((1,), (1,)), ((), ()))
_BM = 1000
_ROWS = pl.BlockSpec((_BM, F), lambda i: (i, 0))
_WMAT = pl.BlockSpec((F, F), lambda i: (0, 0))
_VEC = pl.BlockSpec((1, F), lambda i: (0, 0))


def _proj2(xu, wu, su_, bu_, xm, wm, sm_, bm_):
    def body(xu_ref, wu_ref, su_ref, bu_ref, xm_ref, wm_ref, sm_ref, bm_ref,
             ou_ref, om_ref):
        for x_ref, w_ref, sc_ref, sh_ref, o_ref in (
                (xu_ref, wu_ref, su_ref, bu_ref, ou_ref),
                (xm_ref, wm_ref, sm_ref, bm_ref, om_ref)):
            acc = lax.dot_general(x_ref[...], w_ref[...], _CONTRACT_T,
                                  preferred_element_type=jnp.float32)
            o_ref[...] = jnp.maximum(acc * sc_ref[...] + sh_ref[...], 0.0)

    return pl.pallas_call(
        body,
        grid=(N // _BM,),
        in_specs=[_ROWS, _WMAT, _VEC, _VEC, _ROWS, _WMAT, _VEC, _VEC],
        out_specs=[_ROWS, _ROWS],
        out_shape=[jax.ShapeDtypeStruct((N, F), jnp.float32),
                   jax.ShapeDtypeStruct((N, F), jnp.float32)],
    )(xu, wu, su_, bu_, xm, wm, sm_, bm_)


def _combine2(aggm, cntm, hm, wlm, wrm, bm_, aggu, cntu, hu, wlu, wru, bu_,
              relu):
    def one(a_ref, c_ref, h_ref, wl_ref, wr_ref, b_ref, o_ref):
        r = 1.0 / jnp.maximum(c_ref[:, 0:1], 1.0)
        out = (lax.dot_general(a_ref[...] * r, wl_ref[...], _CONTRACT_T,
                               preferred_element_type=jnp.float32)
               + lax.dot_general(h_ref[...], wr_ref[...], _CONTRACT_T,
                                 preferred_element_type=jnp.float32)
               + b_ref[...])
        o_ref[...] = jnp.maximum(out, 0.0) if relu else out

    def body(am_ref, cm_ref, hm_ref, wlm_ref, wrm_ref, bm_ref,
             au_ref, cu_ref, hu_ref, wlu_ref, wru_ref, bu_ref,
             om_ref, ou_ref):
        one(am_ref, cm_ref, hm_ref, wlm_ref, wrm_ref, bm_ref, om_ref)
        one(au_ref, cu_ref, hu_ref, wlu_ref, wru_ref, bu_ref, ou_ref)

    return pl.pallas_call(
        body,
        grid=(N // _BM,),
        in_specs=[_ROWS, _ROWS, _ROWS, _WMAT, _WMAT, _VEC,
                  _ROWS, _ROWS, _ROWS, _WMAT, _WMAT, _VEC],
        out_specs=[_ROWS, _ROWS],
        out_shape=[jax.ShapeDtypeStruct((N, F), jnp.float32),
                   jax.ShapeDtypeStruct((N, F), jnp.float32)],
    )(aggm, cntm, hm, wlm, wrm, bm_, aggu, cntu, hu, wlu, wru, bu_)


def kernel(x_user, x_movie, edge_index_rates, edge_index_rated_by,
           lin_user_W, lin_user_b, lin_movie_W, lin_movie_b,
           bn_user_g, bn_user_beta, bn_user_m, bn_user_v,
           bn_movie_g, bn_movie_beta, bn_movie_m, bn_movie_v,
           c1_rates_Wl, c1_rates_bl, c1_rates_Wr,
           c1_rb_Wl, c1_rb_bl, c1_rb_Wr,
           c2_rates_Wl, c2_rates_bl, c2_rates_Wr,
           c2_rb_Wl, c2_rb_bl, c2_rb_Wr):
    eps = 1e-5
    su = edge_index_rates[0].astype(jnp.int32)
    dm = edge_index_rates[1].astype(jnp.int32)
    sm = edge_index_rated_by[0].astype(jnp.int32)
    du = edge_index_rated_by[1].astype(jnp.int32)

    scl_u = bn_user_g / jnp.sqrt(bn_user_v + eps)
    sh_u = (lin_user_b - bn_user_m) * scl_u + bn_user_beta
    scl_m = bn_movie_g / jnp.sqrt(bn_movie_v + eps)
    sh_m = (lin_movie_b - bn_movie_m) * scl_m + bn_movie_beta

    hu, hm = _proj2(x_user, lin_user_W, scl_u[None, :], sh_u[None, :],
                    x_movie, lin_movie_W, scl_m[None, :], sh_m[None, :])

    aggm, aggu, cm, cu = _agg_l1(hu, hm, su, dm, sm, du)

    m1, u1 = _combine2(aggm, cm, hm, c1_rates_Wl, c1_rates_Wr,
                       c1_rates_bl[None, :],
                       aggu, cu, hu, c1_rb_Wl, c1_rb_Wr,
                       c1_rb_bl[None, :], True)

    aggm2, aggu2 = _agg_l2(u1, m1, su, dm, sm, du)

    m2, u2 = _combine2(aggm2, cm, m1, c2_rates_Wl, c2_rates_Wr,
                       c2_rates_bl[None, :],
                       aggu2, cu, u1, c2_rb_Wl, c2_rb_Wr,
                       c2_rb_bl[None, :], False)
    return (u2, m2)
